# Initial kernel scaffold; baseline (speedup 1.0000x reference)
import jax
import jax.numpy as jnp
from jax.experimental import pallas as pl


def kernel(x, W, b):
    raise NotImplementedError("write your pallas kernel here")


# speedup vs baseline: 1.0638x; 1.0638x over previous
import functools

import jax
import jax.numpy as jnp
from jax.experimental import pallas as pl


TOPK = 8


def _router_body(nblocks, ntokens, x_ref, wt_ref, b_ref,
                 vals_ref, idx_ref, psum_ref):
    pid = pl.program_id(0)
    logits = jnp.dot(x_ref[...], wt_ref[...],
                     preferred_element_type=jnp.float32)
    logits = logits + b_ref[...]
    m = jnp.max(logits, axis=1, keepdims=True)
    e = jnp.exp(logits - m)
    s = jnp.sum(e, axis=1, keepdims=True)
    p = e / s

    @pl.when(pid == 0)
    def _init():
        psum_ref[...] = jnp.zeros_like(psum_ref)

    psum_ref[...] += jnp.sum(p, axis=0, keepdims=True)

    @pl.when(pid == nblocks - 1)
    def _finish():
        psum_ref[...] *= jnp.float32(1.0 / ntokens)

    E = p.shape[1]
    iota = jax.lax.broadcasted_iota(jnp.int32, p.shape, 1)
    v = p
    val_cols = []
    idx_cols = []
    for _ in range(TOPK):
        mk = jnp.max(v, axis=1, keepdims=True)
        ak = jnp.min(jnp.where(v == mk, iota, E), axis=1, keepdims=True)
        val_cols.append(mk)
        idx_cols.append(ak)
        v = jnp.where(iota == ak, jnp.float32(-1.0), v)
    vals_ref[...] = jnp.concatenate(val_cols, axis=1)
    idx_ref[...] = jnp.concatenate(idx_cols, axis=1)


@functools.partial(jax.jit, static_argnames=("block_rows",))
def _router(x, W, b, block_rows=512):
    B, S, D = x.shape
    E = W.shape[0]
    N = B * S
    R = block_rows
    while N % R:
        R //= 2
    nblocks = N // R

    xf = x.reshape(N, D)
    wt = W.T
    b2 = b.reshape(1, E)

    vals, idx, psum = pl.pallas_call(
        functools.partial(_router_body, nblocks, N),
        grid=(nblocks,),
        in_specs=[
            pl.BlockSpec((R, D), lambda i: (i, 0)),
            pl.BlockSpec((D, E), lambda i: (0, 0)),
            pl.BlockSpec((1, E), lambda i: (0, 0)),
        ],
        out_specs=[
            pl.BlockSpec((R, TOPK), lambda i: (i, 0)),
            pl.BlockSpec((R, TOPK), lambda i: (i, 0)),
            pl.BlockSpec((1, E), lambda i: (0, 0)),
        ],
        out_shape=[
            jax.ShapeDtypeStruct((N, TOPK), jnp.float32),
            jax.ShapeDtypeStruct((N, TOPK), jnp.int32),
            jax.ShapeDtypeStruct((1, E), jnp.float32),
        ],
    )(xf, wt, b2)

    return (vals.reshape(B, S, TOPK), idx.reshape(B, S, TOPK),
            psum.reshape(E))


def kernel(x, W, b):
    return _router(x, W, b)


# speedup vs baseline: 1.6502x; 1.5513x over previous
import functools

import jax
import jax.numpy as jnp
from jax.experimental import pallas as pl
from jax.experimental.pallas import tpu as pltpu


TOPK = 8


def _router_body(nblocks, ntokens, x_ref, w_ref, b_ref,
                 vals_ref, idx_ref, psum_ref, pacc_ref):
    pid = pl.program_id(0)
    lt = jax.lax.dot_general(w_ref[...], x_ref[...],
                             (((1,), (1,)), ((), ())),
                             preferred_element_type=jnp.float32)
    lt = lt + b_ref[...]
    m = jnp.max(lt, axis=0, keepdims=True)
    e = jnp.exp(lt - m)
    s = jnp.sum(e, axis=0, keepdims=True)
    p = e / s

    @pl.when(pid == 0)
    def _init():
        pacc_ref[...] = jnp.zeros_like(pacc_ref)

    pacc_ref[...] += p

    @pl.when(pid == nblocks - 1)
    def _finish():
        psum_ref[...] = (jnp.sum(pacc_ref[...], axis=1, keepdims=True)
                         * jnp.float32(1.0 / ntokens))

    E = p.shape[0]
    iota = jax.lax.broadcasted_iota(jnp.int32, p.shape, 0)
    v = p
    val_rows = []
    idx_rows = []
    for _ in range(TOPK):
        mk = jnp.max(v, axis=0, keepdims=True)
        ak = jnp.min(jnp.where(v == mk, iota, E), axis=0, keepdims=True)
        val_rows.append(mk)
        idx_rows.append(ak)
        v = jnp.where(iota == ak, jnp.float32(-1.0), v)
    vals_ref[...] = jnp.concatenate(val_rows, axis=0)
    idx_ref[...] = jnp.concatenate(idx_rows, axis=0)


@functools.partial(jax.jit, static_argnames=("block_rows",))
def _router(x, W, b, block_rows=512):
    B, S, D = x.shape
    E = W.shape[0]
    N = B * S
    R = block_rows
    while N % R:
        R //= 2
    nblocks = N // R

    xf = x.reshape(N, D)
    b2 = b.reshape(E, 1)

    vals, idx, psum = pl.pallas_call(
        functools.partial(_router_body, nblocks, N),
        grid=(nblocks,),
        in_specs=[
            pl.BlockSpec((R, D), lambda i: (i, 0)),
            pl.BlockSpec((E, D), lambda i: (0, 0)),
            pl.BlockSpec((E, 1), lambda i: (0, 0)),
        ],
        out_specs=[
            pl.BlockSpec((TOPK, R), lambda i: (0, i)),
            pl.BlockSpec((TOPK, R), lambda i: (0, i)),
            pl.BlockSpec((E, 1), lambda i: (0, 0)),
        ],
        out_shape=[
            jax.ShapeDtypeStruct((TOPK, N), jnp.float32),
            jax.ShapeDtypeStruct((TOPK, N), jnp.int32),
            jax.ShapeDtypeStruct((E, 1), jnp.float32),
        ],
        scratch_shapes=[pltpu.VMEM((E, R), jnp.float32)],
    )(xf, W, b2)

    return (vals.T.reshape(B, S, TOPK), idx.T.reshape(B, S, TOPK),
            psum.reshape(E))


def kernel(x, W, b):
    return _router(x, W, b)


# speedup vs baseline: 1.7855x; 1.0820x over previous
import functools

import jax
import jax.numpy as jnp
from jax.experimental import pallas as pl
from jax.experimental.pallas import tpu as pltpu


TOPK = 8


def _router_body(nblocks, ntokens, x_ref, w_ref, b_ref,
                 vals_ref, idx_ref, psum_ref, pacc_ref):
    pid = pl.program_id(0)
    lt = jax.lax.dot_general(w_ref[...], x_ref[...],
                             (((1,), (1,)), ((), ())),
                             preferred_element_type=jnp.float32)
    lt = lt + b_ref[...]
    m = jnp.max(lt, axis=0, keepdims=True)
    e = jnp.exp(lt - m)
    s = jnp.sum(e, axis=0, keepdims=True)
    p = e / s

    @pl.when(pid == 0)
    def _init():
        pacc_ref[...] = jnp.zeros_like(pacc_ref)

    pacc_ref[...] += p

    @pl.when(pid == nblocks - 1)
    def _finish():
        psum_ref[...] = (jnp.sum(pacc_ref[...], axis=1, keepdims=True)
                         * jnp.float32(1.0 / ntokens))

    E = p.shape[0]
    iota = jax.lax.broadcasted_iota(jnp.int32, p.shape, 0)
    v = p
    val_rows = []
    idx_rows = []
    for _ in range(TOPK):
        mk = jnp.max(v, axis=0, keepdims=True)
        ak = jnp.min(jnp.where(v == mk, iota, E), axis=0, keepdims=True)
        val_rows.append(mk)
        idx_rows.append(ak)
        v = jnp.where(iota == ak, jnp.float32(-1.0), v)
    vals_ref[...] = jnp.concatenate(val_rows, axis=0)
    idx_ref[...] = jnp.concatenate(idx_rows, axis=0)


@functools.partial(jax.jit, static_argnames=("block_rows",))
def _router(x, W, b, block_rows=1024):
    B, S, D = x.shape
    E = W.shape[0]
    N = B * S
    R = block_rows
    while N % R:
        R //= 2
    nblocks = N // R

    xf = x.reshape(N, D)
    b2 = b.reshape(E, 1)

    vals, idx, psum = pl.pallas_call(
        functools.partial(_router_body, nblocks, N),
        grid=(nblocks,),
        in_specs=[
            pl.BlockSpec((R, D), lambda i: (i, 0)),
            pl.BlockSpec((E, D), lambda i: (0, 0)),
            pl.BlockSpec((E, 1), lambda i: (0, 0)),
        ],
        out_specs=[
            pl.BlockSpec((TOPK, R), lambda i: (0, i)),
            pl.BlockSpec((TOPK, R), lambda i: (0, i)),
            pl.BlockSpec((E, 1), lambda i: (0, 0)),
        ],
        out_shape=[
            jax.ShapeDtypeStruct((TOPK, N), jnp.float32),
            jax.ShapeDtypeStruct((TOPK, N), jnp.int32),
            jax.ShapeDtypeStruct((E, 1), jnp.float32),
        ],
        scratch_shapes=[pltpu.VMEM((E, R), jnp.float32)],
    )(xf, W, b2)

    return (vals.T.reshape(B, S, TOPK), idx.T.reshape(B, S, TOPK),
            psum.reshape(E))


def kernel(x, W, b):
    return _router(x, W, b)
